# baseline (speedup 1.0000x reference)
import functools
import jax
import jax.numpy as jnp
from jax import lax
from jax.experimental import pallas as pl
from jax.experimental.pallas import tpu as pltpu
from jax.experimental.pallas import tpu_sc as plsc

VOCAB = 100000
NADD = 1024
D = 64
BATCH = 4096
HIST = 200
B = BATCH * HIST
OUTROWS = B + 32
NC, NS = 2, 16
NW = NC * NS
BPW = B // NW
CH = 512
NCHUNK = BPW // CH
NVEC = CH // 16
NBLK = NVEC + 1

_mesh = plsc.VectorSubcoreMesh(core_axis_name="c", subcore_axis_name="s")


@functools.partial(
    pl.kernel,
    mesh=_mesh,
    out_type=jax.ShapeDtypeStruct((OUTROWS, D), jnp.float32),
    scratch_types=[
        pltpu.VMEM((2, CH), jnp.int32),
        pltpu.VMEM((2, CH), jnp.int32),
        pltpu.VMEM((2, NBLK, 16), jnp.int32),
        pltpu.VMEM((2, NBLK, 16), jnp.int32),
        pltpu.VMEM((2, CH, D), jnp.float32),
        pltpu.VMEM((16, D), jnp.float32),
        pltpu.SemaphoreType.DMA,
        pltpu.SemaphoreType.DMA,
        pltpu.SemaphoreType.DMA,
        pltpu.SemaphoreType.DMA,
        pltpu.SemaphoreType.DMA,
        pltpu.SemaphoreType.DMA,
        pltpu.SemaphoreType.DMA,
    ],
    compiler_params=pltpu.CompilerParams(use_tc_tiling_on_sc=False, needs_layout_passes=False),
)
def _gather_kernel(frozen_hbm, train_hbm, idx_hbm, out_hbm,
                   idx_v, fidx_v, pos_v, trn_v, rows_v, stage_v,
                   isem0, isem1, gsem0, gsem1, ssem0, ssem1, fsem):
    wid = lax.axis_index("s") * NC + lax.axis_index("c")
    base = wid * BPW
    isems = (isem0, isem1)
    gsems = (gsem0, gsem1)
    ssems = (ssem0, ssem1)

    def idx_copy(c, b):
        return pltpu.make_async_copy(
            idx_hbm.at[pl.ds(base + c * CH, CH)], idx_v.at[b], isems[b])

    def gather_copy(b):
        return pltpu.make_async_copy(
            frozen_hbm.at[fidx_v.at[b]], rows_v.at[b], gsems[b])

    def store_copy(c, b):
        return pltpu.make_async_copy(
            rows_v.at[b], out_hbm.at[pl.ds(base + c * CH, CH)], ssems[b])

    def vec_pass(c, b):
        pos = base + c * CH + lax.iota(jnp.int32, 16)
        cnt = jnp.int32(0)
        for j in range(NVEC):
            iv = idx_v.at[b][pl.ds(j * 16, 16)]
            m = iv >= VOCAB
            fidx_v.at[b][pl.ds(j * 16, 16)] = jnp.minimum(iv, VOCAB - 1)
            tgt = cnt + plsc.cumsum(m.astype(jnp.int32)) - 1
            row = lax.shift_right_logical(tgt, 4)
            col = lax.bitwise_and(tgt, 15)
            plsc.store_scatter(pos_v.at[b], [row, col], pos, mask=m)
            plsc.store_scatter(trn_v.at[b], [row, col], iv - VOCAB, mask=m)
            cnt = tgt[15] + 1
            pos = pos + 16
        n = cnt
        tgt = n + lax.iota(jnp.int32, 16)
        row = lax.shift_right_logical(tgt, 4)
        col = lax.bitwise_and(tgt, 15)
        plsc.store_scatter(pos_v.at[b], [row, col],
                           jnp.full((16,), B, jnp.int32) + wid)
        plsc.store_scatter(trn_v.at[b], [row, col],
                           jnp.zeros((16,), jnp.int32))
        return n

    def fixup(b, n):
        nblk = lax.shift_right_logical(n + 15, 4)

        def fix_body(s, _):
            pltpu.make_async_copy(
                train_hbm.at[trn_v.at[b].at[s]], stage_v, fsem).start()
            pltpu.make_async_copy(
                train_hbm.at[trn_v.at[b].at[s]], stage_v, fsem).wait()
            pltpu.make_async_copy(
                stage_v, out_hbm.at[pos_v.at[b].at[s]], fsem).start()
            pltpu.make_async_copy(
                stage_v, out_hbm.at[pos_v.at[b].at[s]], fsem).wait()
            return ()

        lax.fori_loop(0, nblk, fix_body, ())

    idx_copy(0, 0).start()
    idx_copy(0, 0).wait()
    n0 = vec_pass(0, 0)
    gather_copy(0).start()
    idx_copy(1, 1).start()
    idx_copy(1, 1).wait()
    n1 = vec_pass(1, 1)
    gather_copy(1).start()
    gather_copy(0).wait()
    store_copy(0, 0).start()
    idx_copy(2, 0).start()

    def pair_body(g, ns):
        n0, n1 = ns
        for b in range(2):
            c = 2 * g + b
            n_prev = n0 if b == 0 else n1
            idx_copy(c, b).wait()
            store_copy(c - 2, b).wait()
            fixup(b, n_prev)
            n_c = vec_pass(c, b)
            gather_copy(b).start()
            gather_copy(1 - b).wait()
            store_copy(c - 1, 1 - b).start()
            nxt = c + 1
            nxt = jnp.where(nxt == NCHUNK, 0, nxt)
            idx_copy(nxt, 1 - b).start()
            if b == 0:
                n0 = n_c
            else:
                n1 = n_c
        return (n0, n1)

    n0, n1 = lax.fori_loop(1, NCHUNK // 2, pair_body, (n0, n1))

    gather_copy(1).wait()
    store_copy(NCHUNK - 1, 1).start()
    store_copy(NCHUNK - 2, 0).wait()
    fixup(0, n0)
    store_copy(NCHUNK - 1, 1).wait()
    fixup(1, n1)
    idx_copy(0, 0).wait()


@jax.jit
def _impl(embed_frozen, weights_train, idx):
    idx2 = idx.reshape(B).astype(jnp.int32)
    out = _gather_kernel(embed_frozen, weights_train, idx2)
    return out[:B].reshape(BATCH, HIST, D)


def kernel(embed_frozen, weights_train, idx):
    return _impl(embed_frozen, weights_train, idx)


# speedup vs baseline: 2.0393x; 2.0393x over previous
import functools
import jax
import jax.numpy as jnp
from jax import lax
from jax.experimental import pallas as pl
from jax.experimental.pallas import tpu as pltpu
from jax.experimental.pallas import tpu_sc as plsc

VOCAB = 100000
NADD = 1024
D = 64
BATCH = 4096
HIST = 200
B = BATCH * HIST
NC, NS = 2, 16
NW = NC * NS
BPW = B // NW
CH = 512
NCHUNK = BPW // CH
GW = 512
NSUB = CH // GW

_mesh = plsc.VectorSubcoreMesh(core_axis_name="c", subcore_axis_name="s")


@functools.partial(
    pl.kernel,
    mesh=_mesh,
    out_type=jax.ShapeDtypeStruct((B, D), jnp.float32),
    scratch_types=[
        pltpu.VMEM((2, CH), jnp.int32),
        pltpu.VMEM((2, CH, D), jnp.float32),
        pltpu.SemaphoreType.DMA,
        pltpu.SemaphoreType.DMA,
        pltpu.SemaphoreType.DMA,
        pltpu.SemaphoreType.DMA,
        pltpu.SemaphoreType.DMA,
        pltpu.SemaphoreType.DMA,
    ],
    compiler_params=pltpu.CompilerParams(use_tc_tiling_on_sc=False),
)
def _gather_kernel(table_hbm, idx_hbm, out_hbm, idx_v, rows_v,
                   isem0, isem1, gsem0, gsem1, ssem0, ssem1):
    wid = lax.axis_index("s") * NC + lax.axis_index("c")
    base = wid * BPW
    isems = (isem0, isem1)
    gsems = (gsem0, gsem1)
    ssems = (ssem0, ssem1)

    def idx_copy(c, b):
        return pltpu.make_async_copy(
            idx_hbm.at[pl.ds(base + c * CH, CH)], idx_v.at[b], isems[b])

    def gather_copy(j, b):
        return pltpu.make_async_copy(
            table_hbm.at[idx_v.at[b].at[pl.ds(j * GW, GW)]],
            rows_v.at[b].at[pl.ds(j * GW, GW)],
            gsems[b])

    def store_copy(c, b):
        return pltpu.make_async_copy(
            rows_v.at[b], out_hbm.at[pl.ds(base + c * CH, CH)], ssems[b])

    def fire_gathers(b):
        for j in range(NSUB):
            gather_copy(j, b).start()

    def drain_gathers(b):
        for j in range(NSUB):
            gather_copy(j, b).wait()

    idx_copy(0, 0).start()
    idx_copy(0, 0).wait()
    fire_gathers(0)
    idx_copy(1, 1).start()
    idx_copy(1, 1).wait()
    fire_gathers(1)
    drain_gathers(0)
    store_copy(0, 0).start()
    idx_copy(2, 0).start()

    def pair_body(g, _):
        for b in range(2):
            c = 2 * g + b
            idx_copy(c, b).wait()
            store_copy(c - 2, b).wait()
            fire_gathers(b)
            drain_gathers(1 - b)
            store_copy(c - 1, 1 - b).start()
            nxt = c + 1
            nxt = jnp.where(nxt == NCHUNK, 0, nxt)
            idx_copy(nxt, 1 - b).start()
        return ()

    lax.fori_loop(1, NCHUNK // 2, pair_body, ())

    drain_gathers(1)
    store_copy(NCHUNK - 1, 1).start()
    store_copy(NCHUNK - 2, 0).wait()
    store_copy(NCHUNK - 1, 1).wait()
    idx_copy(0, 0).wait()


@jax.jit
def _impl(embed_frozen, weights_train, idx):
    table = jnp.concatenate((embed_frozen, weights_train), axis=0)
    idx2 = idx.reshape(B).astype(jnp.int32)
    out = _gather_kernel(table, idx2)
    return out.reshape(BATCH, HIST, D)


def kernel(embed_frozen, weights_train, idx):
    return _impl(embed_frozen, weights_train, idx)


# speedup vs baseline: 2.0415x; 1.0011x over previous
import functools
import jax
import jax.numpy as jnp
from jax import lax
from jax.experimental import pallas as pl
from jax.experimental.pallas import tpu as pltpu
from jax.experimental.pallas import tpu_sc as plsc

VOCAB = 100000
NADD = 1024
D = 64
BATCH = 4096
HIST = 200
B = BATCH * HIST
NC, NS = 2, 16
NW = NC * NS
BPW = B // NW
CH = 640
NCHUNK = BPW // CH
GW = 640
NSUB = CH // GW

_mesh = plsc.VectorSubcoreMesh(core_axis_name="c", subcore_axis_name="s")


@functools.partial(
    pl.kernel,
    mesh=_mesh,
    out_type=jax.ShapeDtypeStruct((B, D), jnp.float32),
    scratch_types=[
        pltpu.VMEM((2, CH), jnp.int32),
        pltpu.VMEM((2, CH, D), jnp.float32),
        pltpu.SemaphoreType.DMA,
        pltpu.SemaphoreType.DMA,
        pltpu.SemaphoreType.DMA,
        pltpu.SemaphoreType.DMA,
        pltpu.SemaphoreType.DMA,
        pltpu.SemaphoreType.DMA,
    ],
    compiler_params=pltpu.CompilerParams(use_tc_tiling_on_sc=False),
)
def _gather_kernel(table_hbm, idx_hbm, out_hbm, idx_v, rows_v,
                   isem0, isem1, gsem0, gsem1, ssem0, ssem1):
    wid = lax.axis_index("s") * NC + lax.axis_index("c")
    base = wid * BPW
    isems = (isem0, isem1)
    gsems = (gsem0, gsem1)
    ssems = (ssem0, ssem1)

    def idx_copy(c, b):
        return pltpu.make_async_copy(
            idx_hbm.at[pl.ds(base + c * CH, CH)], idx_v.at[b], isems[b])

    def gather_copy(j, b):
        return pltpu.make_async_copy(
            table_hbm.at[idx_v.at[b].at[pl.ds(j * GW, GW)]],
            rows_v.at[b].at[pl.ds(j * GW, GW)],
            gsems[b])

    def store_copy(c, b):
        return pltpu.make_async_copy(
            rows_v.at[b], out_hbm.at[pl.ds(base + c * CH, CH)], ssems[b])

    def fire_gathers(b):
        for j in range(NSUB):
            gather_copy(j, b).start()

    def drain_gathers(b):
        for j in range(NSUB):
            gather_copy(j, b).wait()

    idx_copy(0, 0).start()
    idx_copy(0, 0).wait()
    fire_gathers(0)
    idx_copy(1, 1).start()
    idx_copy(1, 1).wait()
    fire_gathers(1)
    drain_gathers(0)
    store_copy(0, 0).start()
    idx_copy(2, 0).start()

    def pair_body(g, _):
        for b in range(2):
            c = 2 * g + b
            idx_copy(c, b).wait()
            store_copy(c - 2, b).wait()
            fire_gathers(b)
            drain_gathers(1 - b)
            store_copy(c - 1, 1 - b).start()
            nxt = c + 1
            nxt = jnp.where(nxt == NCHUNK, 0, nxt)
            idx_copy(nxt, 1 - b).start()
        return ()

    lax.fori_loop(1, NCHUNK // 2, pair_body, ())

    drain_gathers(1)
    store_copy(NCHUNK - 1, 1).start()
    store_copy(NCHUNK - 2, 0).wait()
    store_copy(NCHUNK - 1, 1).wait()
    idx_copy(0, 0).wait()


@jax.jit
def _impl(embed_frozen, weights_train, idx):
    table = jnp.concatenate((embed_frozen, weights_train), axis=0)
    idx2 = idx.reshape(B).astype(jnp.int32)
    out = _gather_kernel(table, idx2)
    return out.reshape(BATCH, HIST, D)


def kernel(embed_frozen, weights_train, idx):
    return _impl(embed_frozen, weights_train, idx)
